# baseline (speedup 1.0000x reference)
import functools

import jax
import jax.numpy as jnp
from jax import lax
from jax.experimental import pallas as pl
from jax.experimental.pallas import tpu as pltpu
from jax.experimental.pallas import tpu_sc as plsc

_BATCH = 4
_SEQ = 8192
_D = 1024
_LANES = 16
_NC = 2
_NS = 16
_NW = _NC * _NS
_ROWS_PER_W = _SEQ // _NW
_C = 8
_CHUNKS = _ROWS_PER_W // _C


def _sc_body(x_hbm, pos_hbm, out_hbm, xbuf, pbuf, sem):
    wid = lax.axis_index("s") * _NC + lax.axis_index("c")
    s_base = wid * _ROWS_PER_W

    def chunk_body(ci, carry):
        s0 = s_base + ci * _C
        pltpu.sync_copy(pos_hbm.at[pl.ds(s0, _C)], pbuf)
        for b in range(_BATCH):
            pltpu.sync_copy(x_hbm.at[b, pl.ds(s0, _C)],
                            xbuf.at[pl.ds(b * _C, _C)])
        for b in range(_BATCH):
            def row_body(r, c2, _b=b):
                rr = _b * _C + r
                for j in range(_D // _LANES):
                    sl = pl.ds(j * _LANES, _LANES)
                    xbuf[rr, sl] = xbuf[rr, sl] + pbuf[r, sl]
                return c2
            lax.fori_loop(0, _C, row_body, 0)
        for b in range(_BATCH):
            pltpu.sync_copy(xbuf.at[pl.ds(b * _C, _C)],
                            out_hbm.at[b, pl.ds(s0, _C)])
        return carry

    lax.fori_loop(0, _CHUNKS, chunk_body, 0)


def kernel(x, pos_weight):
    mesh = plsc.VectorSubcoreMesh(core_axis_name="c", subcore_axis_name="s")
    fn = functools.partial(
        pl.kernel,
        mesh=mesh,
        out_type=jax.ShapeDtypeStruct((_BATCH, _SEQ, _D), jnp.float32),
        scratch_types=[
            pltpu.VMEM((_BATCH * _C, _D), jnp.float32),
            pltpu.VMEM((_C, _D), jnp.float32),
            pltpu.SemaphoreType.DMA,
        ],
    )(_sc_body)
    return fn(x, pos_weight)


# speedup vs baseline: 1.9739x; 1.9739x over previous
import functools

import jax
import jax.numpy as jnp
from jax import lax
from jax.experimental import pallas as pl
from jax.experimental.pallas import tpu as pltpu
from jax.experimental.pallas import tpu_sc as plsc

_BATCH = 4
_SEQ = 8192
_D = 1024
_LANES = 16
_NC = 2
_NS = 16
_NW = _NC * _NS
_ROWS_PER_W = _SEQ // _NW
_C = 8
_CHUNKS = _ROWS_PER_W // _C


def _sc_body(x_hbm, pos_hbm, out_hbm, xbuf, pbuf,
             in0, in1, out0, out1):
    wid = lax.axis_index("s") * _NC + lax.axis_index("c")
    s_base = wid * _ROWS_PER_W
    insem = (in0, in1)
    outsem = (out0, out1)

    def start_in(c, p):
        s0 = s_base + c * _C
        pltpu.async_copy(pos_hbm.at[pl.ds(s0, _C)], pbuf.at[p], insem[p])
        for b in range(_BATCH):
            pltpu.async_copy(x_hbm.at[b, pl.ds(s0, _C)],
                             xbuf.at[p, pl.ds(b * _C, _C)], insem[p])

    def wait_in(p):
        pltpu.make_async_copy(pos_hbm.at[pl.ds(0, _C)], pbuf.at[p],
                              insem[p]).wait()
        pltpu.make_async_copy(x_hbm.at[0, pl.ds(0, _BATCH * _C)], xbuf.at[p],
                              insem[p]).wait()

    def start_out(c, p):
        s0 = s_base + c * _C
        for b in range(_BATCH):
            pltpu.async_copy(xbuf.at[p, pl.ds(b * _C, _C)],
                             out_hbm.at[b, pl.ds(s0, _C)], outsem[p])

    def wait_out(p):
        pltpu.make_async_copy(x_hbm.at[0, pl.ds(0, _BATCH * _C)], xbuf.at[p],
                              outsem[p]).wait()

    def compute(p):
        for b in range(_BATCH):
            def row_body(r, c2, _b=b):
                rr = _b * _C + r
                for j in range(_D // _LANES):
                    sl = pl.ds(j * _LANES, _LANES)
                    xbuf[p, rr, sl] = xbuf[p, rr, sl] + pbuf[p, r, sl]
                return c2
            lax.fori_loop(0, _C, row_body, 0)

    start_in(0, 0)

    def step(si, carry):
        for p in range(2):
            c = si * 2 + p
            q = 1 - p

            @pl.when(c >= 1)
            def _():
                wait_out(q)

            @pl.when(c + 1 < _CHUNKS)
            def _():
                start_in(c + 1, q)

            wait_in(p)
            compute(p)
            start_out(c, p)
        return carry

    lax.fori_loop(0, _CHUNKS // 2, step, 0)
    wait_out(1)


def kernel(x, pos_weight):
    mesh = plsc.VectorSubcoreMesh(core_axis_name="c", subcore_axis_name="s")
    fn = functools.partial(
        pl.kernel,
        mesh=mesh,
        out_type=jax.ShapeDtypeStruct((_BATCH, _SEQ, _D), jnp.float32),
        scratch_types=[
            pltpu.VMEM((2, _BATCH * _C, _D), jnp.float32),
            pltpu.VMEM((2, _C, _D), jnp.float32),
            pltpu.SemaphoreType.DMA,
            pltpu.SemaphoreType.DMA,
            pltpu.SemaphoreType.DMA,
            pltpu.SemaphoreType.DMA,
        ],
    )(_sc_body)
    return fn(x, pos_weight)
